# Initial kernel scaffold; baseline (speedup 1.0000x reference)
import jax
import jax.numpy as jnp
from jax.experimental import pallas as pl


def kernel(x, edge_features, edge_index, Wn, bn, Wet, bet, Wl1, bl1, Wr1, br1, We1, att1, bias1, g1, b1, Wl2, bl2, Wr2, br2, We2, att2, bias2, g2, b2, Wnh, bnh, Weh, beh, Wm1, bm1, Wm2, bm2):
    raise NotImplementedError("write your pallas kernel here")


# speedup vs baseline: 6.8567x; 6.8567x over previous
import functools

import jax
import jax.numpy as jnp
from jax import lax
from jax.experimental import pallas as pl
from jax.experimental.pallas import tpu as pltpu

N = 10000
E = 320000
H = 8
C = 16
D = 128

_BM = 512


def _act(a, act):
    if act is None:
        return a
    if act == "relu":
        return jnp.maximum(a, 0.0)
    if act == "softmax":
        m = jnp.max(a, axis=-1, keepdims=True)
        e = jnp.exp(a - m)
        return e / jnp.sum(e, axis=-1, keepdims=True)
    if act == "sigmoid":
        return 1.0 / (1.0 + jnp.exp(-a))
    raise ValueError(act)


def _tc_linear(x, W, b, act=None, bm=_BM):
    M, K = x.shape
    P = W.shape[1]

    def kern(x_ref, w_ref, b_ref, o_ref):
        a = jnp.dot(x_ref[...], w_ref[...], preferred_element_type=jnp.float32)
        o_ref[...] = _act(a + b_ref[...], act)

    return pl.pallas_call(
        kern,
        grid=(pl.cdiv(M, bm),),
        in_specs=[
            pl.BlockSpec((bm, K), lambda i: (i, 0)),
            pl.BlockSpec((K, P), lambda i: (0, 0)),
            pl.BlockSpec((1, P), lambda i: (0, 0)),
        ],
        out_specs=pl.BlockSpec((bm, P), lambda i: (i, 0)),
        out_shape=jax.ShapeDtypeStruct((M, P), jnp.float32),
    )(x, W, b.reshape(1, P))


def _alpha_call(xls, xrd, ea, We, att):
    bm = _BM
    attf = att.reshape(1, D)

    def kern(xls_ref, xrd_ref, ea_ref, we_ref, att_ref, a_ref, g_ref):
        i = pl.program_id(0)
        m = xls_ref[...] + xrd_ref[...] + jnp.dot(
            ea_ref[...], we_ref[...], preferred_element_type=jnp.float32)
        m = jnp.where(m > 0, m, 0.2 * m) * att_ref[...]
        colh = lax.broadcasted_iota(jnp.int32, (D, H), 0) // C
        hh = lax.broadcasted_iota(jnp.int32, (D, H), 1)
        S = (colh == hh).astype(jnp.float32)
        a = jnp.dot(m, S, preferred_element_type=jnp.float32)
        a_ref[...] = a
        gm = jnp.max(a, axis=0)
        g2 = jnp.concatenate([gm, gm]).reshape(1, 2 * H)

        @pl.when(i == 0)
        def _():
            g_ref[...] = g2

        g_ref[...] = jnp.maximum(g_ref[...], g2)

    return pl.pallas_call(
        kern,
        grid=(pl.cdiv(E, bm),),
        in_specs=[
            pl.BlockSpec((bm, D), lambda i: (i, 0)),
            pl.BlockSpec((bm, D), lambda i: (i, 0)),
            pl.BlockSpec((bm, C), lambda i: (i, 0)),
            pl.BlockSpec((C, D), lambda i: (0, 0)),
            pl.BlockSpec((1, D), lambda i: (0, 0)),
        ],
        out_specs=[
            pl.BlockSpec((bm, H), lambda i: (i, 0)),
            pl.BlockSpec((1, 2 * H), lambda i: (0, 0)),
        ],
        out_shape=[
            jax.ShapeDtypeStruct((E, H), jnp.float32),
            jax.ShapeDtypeStruct((1, 2 * H), jnp.float32),
        ],
    )(xls, xrd, ea, We, attf)


def _ex_call(alpha, gmax):
    bm = 2048

    def kern(a_ref, g_ref, o_ref):
        o_ref[...] = jnp.exp(a_ref[...] - g_ref[...][:, :H])

    return pl.pallas_call(
        kern,
        grid=(pl.cdiv(E, bm),),
        in_specs=[
            pl.BlockSpec((bm, H), lambda i: (i, 0)),
            pl.BlockSpec((1, 2 * H), lambda i: (0, 0)),
        ],
        out_specs=pl.BlockSpec((bm, H), lambda i: (i, 0)),
        out_shape=jax.ShapeDtypeStruct((E, H), jnp.float32),
    )(alpha, gmax)


def _rec_call(parts):
    P = parts.shape[0]
    bn = 2000

    def kern(p_ref, o_ref):
        s = jnp.sum(p_ref[...], axis=0)
        o_ref[...] = 1.0 / (s + 1e-16)

    return pl.pallas_call(
        kern,
        grid=(pl.cdiv(N, bn),),
        in_specs=[pl.BlockSpec((P, bn, H), lambda i: (0, i, 0))],
        out_specs=pl.BlockSpec((bn, H), lambda i: (i, 0)),
        out_shape=jax.ShapeDtypeStruct((N, H), jnp.float32),
    )(parts)


def _msgw_call(xls, ex, recd):
    bm = _BM

    def kern(xls_ref, ex_ref, recd_ref, o_ref):
        w = ex_ref[...] * recd_ref[...]
        colh = lax.broadcasted_iota(jnp.int32, (H, D), 1) // C
        hh = lax.broadcasted_iota(jnp.int32, (H, D), 0)
        ST = (colh == hh).astype(jnp.float32)
        wexp = jnp.dot(w, ST, preferred_element_type=jnp.float32)
        o_ref[...] = xls_ref[...] * wexp

    return pl.pallas_call(
        kern,
        grid=(pl.cdiv(E, bm),),
        in_specs=[
            pl.BlockSpec((bm, D), lambda i: (i, 0)),
            pl.BlockSpec((bm, H), lambda i: (i, 0)),
            pl.BlockSpec((bm, H), lambda i: (i, 0)),
        ],
        out_specs=pl.BlockSpec((bm, D), lambda i: (i, 0)),
        out_shape=jax.ShapeDtypeStruct((E, D), jnp.float32),
    )(xls, ex, recd)


def _combine_ln_call(parts, bias, g, b, res=None):
    P = parts.shape[0]
    bn = 2000
    have_res = res is not None

    def kern(*refs):
        if have_res:
            p_ref, bias_ref, g_ref, b_ref, res_ref, o_ref = refs
        else:
            p_ref, bias_ref, g_ref, b_ref, o_ref = refs
        hsum = jnp.sum(p_ref[...], axis=0) + bias_ref[...]
        mu = jnp.mean(hsum, axis=-1, keepdims=True)
        var = jnp.mean((hsum - mu) ** 2, axis=-1, keepdims=True)
        hn = (hsum - mu) / jnp.sqrt(var + 1e-5) * g_ref[...] + b_ref[...]
        hn = jnp.maximum(hn, 0.0)
        if have_res:
            hn = hn + res_ref[...]
        o_ref[...] = hn

    in_specs = [
        pl.BlockSpec((P, bn, D), lambda i: (0, i, 0)),
        pl.BlockSpec((1, D), lambda i: (0, 0)),
        pl.BlockSpec((1, D), lambda i: (0, 0)),
        pl.BlockSpec((1, D), lambda i: (0, 0)),
    ]
    args = [parts, bias.reshape(1, D), g.reshape(1, D), b.reshape(1, D)]
    if have_res:
        in_specs.append(pl.BlockSpec((bn, D), lambda i: (i, 0)))
        args.append(res)
    return pl.pallas_call(
        kern,
        grid=(pl.cdiv(N, bn),),
        in_specs=in_specs,
        out_specs=pl.BlockSpec((bn, D), lambda i: (i, 0)),
        out_shape=jax.ShapeDtypeStruct((N, D), jnp.float32),
    )(*args)


def _edge_head_call(hs, hd, Weh, beh, Wm1, bm1, Wm2, bm2):
    bm = _BM
    Wm1a = Wm1[:D]
    Wm1b = Wm1[D:]

    def kern(hs_ref, hd_ref, weh_ref, beh_ref, w1a_ref, w1b_ref, b1_ref,
             w2_ref, b2_ref, et_ref, ep_ref):
        hs_ = hs_ref[...]
        hd_ = hd_ref[...]
        et = jnp.dot(hs_, weh_ref[...], preferred_element_type=jnp.float32)
        et_ref[...] = _act(et + beh_ref[...], "softmax")
        hid = jnp.dot(hs_, w1a_ref[...], preferred_element_type=jnp.float32)
        hid = hid + jnp.dot(hd_, w1b_ref[...], preferred_element_type=jnp.float32)
        hid = jnp.maximum(hid + b1_ref[...], 0.0)
        ep = jnp.dot(hid, w2_ref[...], preferred_element_type=jnp.float32)
        ep_ref[...] = _act(ep + b2_ref[...], "sigmoid")

    return pl.pallas_call(
        kern,
        grid=(pl.cdiv(E, bm),),
        in_specs=[
            pl.BlockSpec((bm, D), lambda i: (i, 0)),
            pl.BlockSpec((bm, D), lambda i: (i, 0)),
            pl.BlockSpec((D, 6), lambda i: (0, 0)),
            pl.BlockSpec((1, 6), lambda i: (0, 0)),
            pl.BlockSpec((D, D), lambda i: (0, 0)),
            pl.BlockSpec((D, D), lambda i: (0, 0)),
            pl.BlockSpec((1, D), lambda i: (0, 0)),
            pl.BlockSpec((D, 1), lambda i: (0, 0)),
            pl.BlockSpec((1, 1), lambda i: (0, 0)),
        ],
        out_specs=[
            pl.BlockSpec((bm, 6), lambda i: (i, 0)),
            pl.BlockSpec((bm, 1), lambda i: (i, 0)),
        ],
        out_shape=[
            jax.ShapeDtypeStruct((E, 6), jnp.float32),
            jax.ShapeDtypeStruct((E, 1), jnp.float32),
        ],
    )(hs, hd, Weh, beh.reshape(1, 6), Wm1a, Wm1b, bm1.reshape(1, D),
      Wm2, bm2.reshape(1, 1))


def _gat_layer(h, src, dst, ea, Wl, bl, Wr, br, We, att, bias, g, bln, res):
    xl = _tc_linear(h, Wl, bl)
    xr = _tc_linear(h, Wr, br)
    xls = jnp.take(xl, src, axis=0)
    xrd = jnp.take(xr, dst, axis=0)
    alpha, gmax = _alpha_call(xls, xrd, ea, We, att)
    ex = _ex_call(alpha, gmax)
    denom = jax.ops.segment_sum(ex, dst, num_segments=N)
    rec = _rec_call(denom[None])
    recd = jnp.take(rec, dst, axis=0)
    msg = _msgw_call(xls, ex, recd)
    out = jax.ops.segment_sum(msg, dst, num_segments=N)
    return _combine_ln_call(out[None], bias, g, bln, res)


def kernel(x, edge_features, edge_index, Wn, bn, Wet, bet, Wl1, bl1, Wr1, br1,
           We1, att1, bias1, g1, b1, Wl2, bl2, Wr2, br2, We2, att2, bias2, g2,
           b2, Wnh, bnh, Weh, beh, Wm1, bm1, Wm2, bm2):
    src = edge_index[0]
    dst = edge_index[1]
    ea = _tc_linear(edge_features, Wet, bet)
    h0 = _tc_linear(x, Wn, bn)
    h1 = _gat_layer(h0, src, dst, ea, Wl1, bl1, Wr1, br1, We1, att1, bias1,
                    g1, b1, None)
    h = _gat_layer(h1, src, dst, ea, Wl2, bl2, Wr2, br2, We2, att2, bias2,
                   g2, b2, h0)
    node_type_preds = _tc_linear(h, Wnh, bnh, act="softmax")
    hs = jnp.take(h, src, axis=0)
    hd = jnp.take(h, dst, axis=0)
    edge_type_preds, edge_existence_preds = _edge_head_call(
        hs, hd, Weh, beh, Wm1, bm1, Wm2, bm2)
    return node_type_preds, edge_type_preds, edge_existence_preds


# speedup vs baseline: 18.2650x; 2.6638x over previous
import functools

import jax
import jax.numpy as jnp
from jax import lax
from jax.experimental import pallas as pl
from jax.experimental.pallas import tpu as pltpu
from jax.experimental.pallas import tpu_sc as plsc

N = 10000
E = 320000
H = 8
C = 16
D = 128

_BM = 512

_NC = 2
_NS = 16
_NW = _NC * _NS
_UNITS = E // 128
_WHI = 24
_U_HI = 80
_U_LO = 72
_TAIL = _UNITS - (_WHI * _U_HI + (_NW - _WHI) * _U_LO)
_MAXU = _U_HI
_NPT = 640


def _worker_span(w):
    nu = jnp.where(w < _WHI, _U_HI, _U_LO) + jnp.where(w == _NW - 1, _TAIL, 0)
    ru = jnp.where(w < _WHI, _U_HI * w, _WHI * _U_HI + _U_LO * (w - _WHI))
    return nu, ru


def _load_idx(idx_hbm, idxbuf, w, ru):
    pltpu.sync_copy(idx_hbm.at[pl.ds(ru, _U_LO)], idxbuf.at[pl.ds(0, _U_LO)])

    @pl.when(w < _WHI)
    def _():
        pltpu.sync_copy(idx_hbm.at[pl.ds(ru + _U_LO, _U_HI - _U_LO)],
                        idxbuf.at[pl.ds(_U_LO, _U_HI - _U_LO)])

    @pl.when(w == _NW - 1)
    def _():
        pltpu.sync_copy(idx_hbm.at[pl.ds(_UNITS - _TAIL, _TAIL)],
                        idxbuf.at[pl.ds(_U_LO, _TAIL)])


def _sc_gather(table, idx2d, P):
    mesh = plsc.VectorSubcoreMesh(core_axis_name="c", subcore_axis_name="s")

    @functools.partial(
        pl.kernel,
        out_type=jax.ShapeDtypeStruct((E, P), jnp.float32),
        mesh=mesh,
        scratch_types=[
            pltpu.VMEM((_MAXU, 128), jnp.int32),
            pltpu.VMEM((128, P), jnp.float32),
            pltpu.SemaphoreType.DMA,
        ],
    )
    def k(table_hbm, idx_hbm, out_hbm, idxbuf, rows, sem):
        w = lax.axis_index("s") * _NC + lax.axis_index("c")
        nu, ru = _worker_span(w)
        _load_idx(idx_hbm, idxbuf, w, ru)

        def body(u, _):
            pltpu.async_copy(table_hbm.at[idxbuf.at[u]], rows, sem).wait()
            pltpu.sync_copy(rows, out_hbm.at[pl.ds((ru + u) * 128, 128)])
            return 0

        lax.fori_loop(0, nu, body, 0)

    return k(table, idx2d)


def _sc_scatter(vals, idx2d, zrows, P):
    mesh = plsc.VectorSubcoreMesh(core_axis_name="c", subcore_axis_name="s")

    @functools.partial(
        pl.kernel,
        out_type=jax.ShapeDtypeStruct((2 * N, P), jnp.float32),
        mesh=mesh,
        scratch_types=[
            pltpu.VMEM((_MAXU, 128), jnp.int32),
            pltpu.VMEM((128, P), jnp.float32),
            pltpu.VMEM_SHARED((N, P), jnp.float32),
            pltpu.SemaphoreType.DMA,
        ],
    )
    def k(vals_hbm, idx_hbm, z_hbm, out_hbm, idxbuf, vbuf, acc, sem):
        cid = lax.axis_index("c")
        sid = lax.axis_index("s")
        w = sid * _NC + cid
        nu, ru = _worker_span(w)
        _load_idx(idx_hbm, idxbuf, w, ru)

        @pl.when(sid < _NS - 1)
        def _():
            pltpu.sync_copy(z_hbm, acc.at[pl.ds(sid * _NPT, _NPT)])

        @pl.when(sid == _NS - 1)
        def _():
            pltpu.sync_copy(z_hbm.at[pl.ds(0, N - (_NS - 1) * _NPT)],
                            acc.at[pl.ds((_NS - 1) * _NPT,
                                         N - (_NS - 1) * _NPT)])

        plsc.subcore_barrier()

        def body(u, _):
            pltpu.sync_copy(vals_hbm.at[pl.ds((ru + u) * 128, 128)], vbuf)
            pltpu.sync_copy(vbuf, acc.at[idxbuf.at[u]], add=True)
            return 0

        lax.fori_loop(0, nu, body, 0)
        plsc.subcore_barrier()

        @pl.when(sid < _NS - 1)
        def _():
            pltpu.sync_copy(acc.at[pl.ds(sid * _NPT, _NPT)],
                            out_hbm.at[pl.ds(cid * N + sid * _NPT, _NPT)])

        @pl.when(sid == _NS - 1)
        def _():
            pltpu.sync_copy(
                acc.at[pl.ds((_NS - 1) * _NPT, N - (_NS - 1) * _NPT)],
                out_hbm.at[pl.ds(cid * N + (_NS - 1) * _NPT,
                                 N - (_NS - 1) * _NPT)])

    return k(vals, idx2d, zrows).reshape(2, N, P)


def _act(a, act):
    if act is None:
        return a
    if act == "relu":
        return jnp.maximum(a, 0.0)
    if act == "softmax":
        m = jnp.max(a, axis=-1, keepdims=True)
        e = jnp.exp(a - m)
        return e / jnp.sum(e, axis=-1, keepdims=True)
    if act == "sigmoid":
        return 1.0 / (1.0 + jnp.exp(-a))
    raise ValueError(act)


def _tc_linear(x, W, b, act=None, bm=_BM):
    M, K = x.shape
    P = W.shape[1]

    def kern(x_ref, w_ref, b_ref, o_ref):
        a = jnp.dot(x_ref[...], w_ref[...], preferred_element_type=jnp.float32)
        o_ref[...] = _act(a + b_ref[...], act)

    return pl.pallas_call(
        kern,
        grid=(pl.cdiv(M, bm),),
        in_specs=[
            pl.BlockSpec((bm, K), lambda i: (i, 0)),
            pl.BlockSpec((K, P), lambda i: (0, 0)),
            pl.BlockSpec((1, P), lambda i: (0, 0)),
        ],
        out_specs=pl.BlockSpec((bm, P), lambda i: (i, 0)),
        out_shape=jax.ShapeDtypeStruct((M, P), jnp.float32),
    )(x, W, b.reshape(1, P))


def _alpha_call(xls, xrd, ea, We, att):
    bm = _BM
    attf = att.reshape(1, D)

    def kern(xls_ref, xrd_ref, ea_ref, we_ref, att_ref, a_ref, g_ref):
        i = pl.program_id(0)
        m = xls_ref[...] + xrd_ref[...] + jnp.dot(
            ea_ref[...], we_ref[...], preferred_element_type=jnp.float32)
        m = jnp.where(m > 0, m, 0.2 * m) * att_ref[...]
        colh = lax.broadcasted_iota(jnp.int32, (D, H), 0) // C
        hh = lax.broadcasted_iota(jnp.int32, (D, H), 1)
        S = (colh == hh).astype(jnp.float32)
        a = jnp.dot(m, S, preferred_element_type=jnp.float32)
        a_ref[...] = a
        gm = jnp.max(a, axis=0)
        g2 = jnp.concatenate([gm, gm]).reshape(1, 2 * H)

        @pl.when(i == 0)
        def _():
            g_ref[...] = g2

        g_ref[...] = jnp.maximum(g_ref[...], g2)

    return pl.pallas_call(
        kern,
        grid=(pl.cdiv(E, bm),),
        in_specs=[
            pl.BlockSpec((bm, D), lambda i: (i, 0)),
            pl.BlockSpec((bm, D), lambda i: (i, 0)),
            pl.BlockSpec((bm, C), lambda i: (i, 0)),
            pl.BlockSpec((C, D), lambda i: (0, 0)),
            pl.BlockSpec((1, D), lambda i: (0, 0)),
        ],
        out_specs=[
            pl.BlockSpec((bm, H), lambda i: (i, 0)),
            pl.BlockSpec((1, 2 * H), lambda i: (0, 0)),
        ],
        out_shape=[
            jax.ShapeDtypeStruct((E, H), jnp.float32),
            jax.ShapeDtypeStruct((1, 2 * H), jnp.float32),
        ],
    )(xls, xrd, ea, We, attf)


def _msgw_call(xls, alpha, gmax):
    bm = _BM

    def kern(xls_ref, a_ref, g_ref, o_ref):
        w = jnp.exp(a_ref[...] - g_ref[...][:, :H])
        colh = lax.broadcasted_iota(jnp.int32, (H, D), 1) // C
        hh = lax.broadcasted_iota(jnp.int32, (H, D), 0)
        ST = (colh == hh).astype(jnp.float32)
        wexp = jnp.dot(w, ST, preferred_element_type=jnp.float32)
        o_ref[...] = xls_ref[...] * wexp

    return pl.pallas_call(
        kern,
        grid=(pl.cdiv(E, bm),),
        in_specs=[
            pl.BlockSpec((bm, D), lambda i: (i, 0)),
            pl.BlockSpec((bm, H), lambda i: (i, 0)),
            pl.BlockSpec((1, 2 * H), lambda i: (0, 0)),
        ],
        out_specs=pl.BlockSpec((bm, D), lambda i: (i, 0)),
        out_shape=jax.ShapeDtypeStruct((E, D), jnp.float32),
    )(xls, alpha, gmax)


_AROW = 640


def _sc_scatter_heads(alpha_flat, g16, idx2d, z128, rowiota):
    mesh = plsc.VectorSubcoreMesh(core_axis_name="c", subcore_axis_name="s")

    @functools.partial(
        pl.kernel,
        out_type=jax.ShapeDtypeStruct((_NC, N * H // 128, 128), jnp.float32),
        mesh=mesh,
        compiler_params=pltpu.CompilerParams(needs_layout_passes=False),
        scratch_types=[
            pltpu.VMEM((_MAXU, 128), jnp.int32),
            pltpu.VMEM((128 * H,), jnp.float32),
            pltpu.VMEM((16,), jnp.float32),
            pltpu.VMEM((_AROW, 128), jnp.float32),
            pltpu.VMEM((_AROW // 128, 128), jnp.int32),
            pltpu.VMEM_SHARED((_AROW, 128), jnp.float32),
            pltpu.SemaphoreType.DMA,
        ],
    )
    def k(a_hbm, g_hbm, idx_hbm, z_hbm, ri_hbm, out_hbm, idxbuf, abuf, gbuf,
          acc, riota, shacc, sem):
        cid = lax.axis_index("c")
        sid = lax.axis_index("s")
        w = sid * _NC + cid
        nu, ru = _worker_span(w)
        _load_idx(idx_hbm, idxbuf, w, ru)
        pltpu.sync_copy(z_hbm, acc)
        nz = _AROW // _NS
        pltpu.sync_copy(z_hbm.at[pl.ds(0, nz)],
                        shacc.at[pl.ds(sid * nz, nz)])
        pltpu.sync_copy(ri_hbm, riota)
        pltpu.sync_copy(g_hbm, gbuf)
        io = lax.iota(jnp.int32, 16)
        mlo = io < 8
        mhi = jnp.logical_not(mlo)

        def unit(u, _):
            pltpu.sync_copy(a_hbm.at[pl.ds((ru + u) * 128 * H, 128 * H)], abuf)

            def grp(j16, _2):
                dvec = idxbuf[u, pl.ds(j16 * 16, 16)]
                for p in range(8):
                    ex = jnp.exp(abuf[pl.ds((j16 * 8 + p) * 16, 16)]
                                 - gbuf[...])
                    d0 = dvec[2 * p]
                    d1 = dvec[2 * p + 1]
                    addr = jnp.where(mlo, d0 * H + io, d1 * H + (io - 8))
                    arow = lax.shift_right_logical(addr, 7)
                    acol = jnp.bitwise_and(addr, 127)
                    plsc.addupdate_scatter(acc, [arow, acol], ex, mask=mlo)
                    plsc.addupdate_scatter(acc, [arow, acol], ex, mask=mhi)
                return _2

            lax.fori_loop(0, 8, grp, 0)
            return _

        lax.fori_loop(0, nu, unit, 0)
        plsc.subcore_barrier()

        def red(kk, _):
            pltpu.sync_copy(acc.at[pl.ds(kk * 128, 128)],
                            shacc.at[riota.at[kk]], add=True)
            return 0

        lax.fori_loop(0, _AROW // 128, red, 0)
        plsc.subcore_barrier()

        @pl.when(sid < _NS - 1)
        def _():
            pltpu.sync_copy(shacc.at[pl.ds(sid * 40, 40)],
                            out_hbm.at[cid, pl.ds(sid * 40, 40)])

        @pl.when(sid == _NS - 1)
        def _():
            pltpu.sync_copy(shacc.at[pl.ds(600, 25)],
                            out_hbm.at[cid, pl.ds(600, 25)])

    out = k(alpha_flat, g16, idx2d, z128, rowiota)
    return out.reshape(_NC, N, H)


def _combine_ln_call(parts, dparts, bias, g, b, res=None):
    P = parts.shape[0]
    PD = dparts.shape[0]
    bn = 2048
    have_res = res is not None

    def kern(*refs):
        if have_res:
            p_ref, dp_ref, bias_ref, g_ref, b_ref, res_ref, o_ref = refs
        else:
            p_ref, dp_ref, bias_ref, g_ref, b_ref, o_ref = refs
        den = jnp.sum(dp_ref[...], axis=0)
        rec = 1.0 / (den + 1e-16)
        colh = lax.broadcasted_iota(jnp.int32, (H, D), 1) // C
        hh = lax.broadcasted_iota(jnp.int32, (H, D), 0)
        ST = (colh == hh).astype(jnp.float32)
        recx = jnp.dot(rec, ST, preferred_element_type=jnp.float32)
        hsum = jnp.sum(p_ref[...], axis=0) * recx + bias_ref[...]
        mu = jnp.mean(hsum, axis=-1, keepdims=True)
        var = jnp.mean((hsum - mu) ** 2, axis=-1, keepdims=True)
        hn = (hsum - mu) / jnp.sqrt(var + 1e-5) * g_ref[...] + b_ref[...]
        hn = jnp.maximum(hn, 0.0)
        if have_res:
            hn = hn + res_ref[...]
        o_ref[...] = hn

    in_specs = [
        pl.BlockSpec((P, bn, D), lambda i: (0, i, 0)),
        pl.BlockSpec((PD, bn, H), lambda i: (0, i, 0)),
        pl.BlockSpec((1, D), lambda i: (0, 0)),
        pl.BlockSpec((1, D), lambda i: (0, 0)),
        pl.BlockSpec((1, D), lambda i: (0, 0)),
    ]
    args = [parts, dparts, bias.reshape(1, D), g.reshape(1, D),
            b.reshape(1, D)]
    if have_res:
        in_specs.append(pl.BlockSpec((bn, D), lambda i: (i, 0)))
        args.append(res)
    return pl.pallas_call(
        kern,
        grid=(pl.cdiv(N, bn),),
        in_specs=in_specs,
        out_specs=pl.BlockSpec((bn, D), lambda i: (i, 0)),
        out_shape=jax.ShapeDtypeStruct((N, D), jnp.float32),
    )(*args)


def _edge_head_call(hs, hd, Weh, beh, Wm1, bm1, Wm2, bm2):
    bm = _BM
    Wm1a = Wm1[:D]
    Wm1b = Wm1[D:]

    def kern(hs_ref, hd_ref, weh_ref, beh_ref, w1a_ref, w1b_ref, b1_ref,
             w2_ref, b2_ref, et_ref, ep_ref):
        hs_ = hs_ref[...]
        hd_ = hd_ref[...]
        et = jnp.dot(hs_, weh_ref[...], preferred_element_type=jnp.float32)
        et_ref[...] = _act(et + beh_ref[...], "softmax")
        hid = jnp.dot(hs_, w1a_ref[...], preferred_element_type=jnp.float32)
        hid = hid + jnp.dot(hd_, w1b_ref[...], preferred_element_type=jnp.float32)
        hid = jnp.maximum(hid + b1_ref[...], 0.0)
        ep = jnp.dot(hid, w2_ref[...], preferred_element_type=jnp.float32)
        ep_ref[...] = _act(ep + b2_ref[...], "sigmoid")

    return pl.pallas_call(
        kern,
        grid=(pl.cdiv(E, bm),),
        in_specs=[
            pl.BlockSpec((bm, D), lambda i: (i, 0)),
            pl.BlockSpec((bm, D), lambda i: (i, 0)),
            pl.BlockSpec((D, 6), lambda i: (0, 0)),
            pl.BlockSpec((1, 6), lambda i: (0, 0)),
            pl.BlockSpec((D, D), lambda i: (0, 0)),
            pl.BlockSpec((D, D), lambda i: (0, 0)),
            pl.BlockSpec((1, D), lambda i: (0, 0)),
            pl.BlockSpec((D, 1), lambda i: (0, 0)),
            pl.BlockSpec((1, 1), lambda i: (0, 0)),
        ],
        out_specs=[
            pl.BlockSpec((bm, 6), lambda i: (i, 0)),
            pl.BlockSpec((bm, 1), lambda i: (i, 0)),
        ],
        out_shape=[
            jax.ShapeDtypeStruct((E, 6), jnp.float32),
            jax.ShapeDtypeStruct((E, 1), jnp.float32),
        ],
    )(hs, hd, Weh, beh.reshape(1, 6), Wm1a, Wm1b, bm1.reshape(1, D),
      Wm2, bm2.reshape(1, 1))


def _gat_layer(h, src2d, dst2d, ea, Wl, bl, Wr, br, We, att, bias, g, bln,
               res, rowiota, z128):
    xl = _tc_linear(h, Wl, bl)
    xr = _tc_linear(h, Wr, br)
    xls = _sc_gather(xl, src2d, D)
    xrd = _sc_gather(xr, dst2d, D)
    alpha, gmax = _alpha_call(xls, xrd, ea, We, att)
    denom_parts = _sc_scatter_heads(alpha.reshape(-1), gmax.reshape(-1),
                                    dst2d, z128, rowiota)
    msg = _msgw_call(xls, alpha, gmax)
    out_parts = _sc_scatter(msg, dst2d, z128, D)
    return _combine_ln_call(out_parts, denom_parts, bias, g, bln, res)


def kernel(x, edge_features, edge_index, Wn, bn, Wet, bet, Wl1, bl1, Wr1, br1,
           We1, att1, bias1, g1, b1, Wl2, bl2, Wr2, br2, We2, att2, bias2, g2,
           b2, Wnh, bnh, Weh, beh, Wm1, bm1, Wm2, bm2):
    src2d = edge_index[0].reshape(_UNITS, 128)
    dst2d = edge_index[1].reshape(_UNITS, 128)
    rowiota = jnp.arange(_AROW, dtype=jnp.int32).reshape(_AROW // 128, 128)
    z128 = jnp.zeros((_NPT, D), jnp.float32)
    ea = _tc_linear(edge_features, Wet, bet)
    h0 = _tc_linear(x, Wn, bn)
    h1 = _gat_layer(h0, src2d, dst2d, ea, Wl1, bl1, Wr1, br1, We1, att1,
                    bias1, g1, b1, None, rowiota, z128)
    h = _gat_layer(h1, src2d, dst2d, ea, Wl2, bl2, Wr2, br2, We2, att2,
                   bias2, g2, b2, h0, rowiota, z128)
    node_type_preds = _tc_linear(h, Wnh, bnh, act="softmax")
    hs = _sc_gather(h, src2d, D)
    hd = _sc_gather(h, dst2d, D)
    edge_type_preds, edge_existence_preds = _edge_head_call(
        hs, hd, Weh, beh, Wm1, bm1, Wm2, bm2)
    return node_type_preds, edge_type_preds, edge_existence_preds


# speedup vs baseline: 19.1010x; 1.0458x over previous
import functools

import jax
import jax.numpy as jnp
from jax import lax
from jax.experimental import pallas as pl
from jax.experimental.pallas import tpu as pltpu
from jax.experimental.pallas import tpu_sc as plsc

N = 10000
E = 320000
H = 8
C = 16
D = 128

_BM = 512

_NC = 2
_NS = 16
_NW = _NC * _NS
_UNITS = E // 128
_WHI = 24
_U_HI = 80
_U_LO = 72
_TAIL = _UNITS - (_WHI * _U_HI + (_NW - _WHI) * _U_LO)
_MAXU = _U_HI
_GRP = 4
_GRPS = 2
_NPT = 640


def _worker_span(w):
    nu = jnp.where(w < _WHI, _U_HI, _U_LO) + jnp.where(w == _NW - 1, _TAIL, 0)
    ru = jnp.where(w < _WHI, _U_HI * w, _WHI * _U_HI + _U_LO * (w - _WHI))
    return nu, ru


def _load_idx(idx_hbm, idxbuf, w, ru):
    pltpu.sync_copy(idx_hbm.at[pl.ds(ru, _U_LO)], idxbuf.at[pl.ds(0, _U_LO)])

    @pl.when(w < _WHI)
    def _():
        pltpu.sync_copy(idx_hbm.at[pl.ds(ru + _U_LO, _U_HI - _U_LO)],
                        idxbuf.at[pl.ds(_U_LO, _U_HI - _U_LO)])

    @pl.when(w == _NW - 1)
    def _():
        pltpu.sync_copy(idx_hbm.at[pl.ds(_UNITS - _TAIL, _TAIL)],
                        idxbuf.at[pl.ds(_U_LO, _TAIL)])


def _sc_gather(table, idx2d, P):
    mesh = plsc.VectorSubcoreMesh(core_axis_name="c", subcore_axis_name="s")

    @functools.partial(
        pl.kernel,
        out_type=jax.ShapeDtypeStruct((E, P), jnp.float32),
        mesh=mesh,
        scratch_types=[
            pltpu.VMEM((_MAXU, 128), jnp.int32),
            pltpu.VMEM((_GRP * 128, P), jnp.float32),
            pltpu.SemaphoreType.DMA,
        ],
    )
    def k(table_hbm, idx_hbm, out_hbm, idxbuf, rows, sem):
        w = lax.axis_index("s") * _NC + lax.axis_index("c")
        nu, ru = _worker_span(w)
        _load_idx(idx_hbm, idxbuf, w, ru)

        def body(g, _):
            u0 = g * _GRP
            cps = [
                pltpu.async_copy(table_hbm.at[idxbuf.at[u0 + j]],
                                 rows.at[pl.ds(j * 128, 128)], sem)
                for j in range(_GRP)
            ]
            for cp in cps:
                cp.wait()
            pltpu.sync_copy(rows,
                            out_hbm.at[pl.ds((ru + u0) * 128, _GRP * 128)])
            return 0

        lax.fori_loop(0, nu // _GRP, body, 0)

    return k(table, idx2d)


def _sc_scatter(vals, idx2d, zrows, P):
    mesh = plsc.VectorSubcoreMesh(core_axis_name="c", subcore_axis_name="s")

    @functools.partial(
        pl.kernel,
        out_type=jax.ShapeDtypeStruct((2 * N, P), jnp.float32),
        mesh=mesh,
        scratch_types=[
            pltpu.VMEM((_MAXU, 128), jnp.int32),
            pltpu.VMEM((_GRPS * 128, P), jnp.float32),
            pltpu.VMEM_SHARED((N, P), jnp.float32),
            pltpu.SemaphoreType.DMA,
        ],
    )
    def k(vals_hbm, idx_hbm, z_hbm, out_hbm, idxbuf, vbuf, acc, sem):
        cid = lax.axis_index("c")
        sid = lax.axis_index("s")
        w = sid * _NC + cid
        nu, ru = _worker_span(w)
        _load_idx(idx_hbm, idxbuf, w, ru)

        @pl.when(sid < _NS - 1)
        def _():
            pltpu.sync_copy(z_hbm, acc.at[pl.ds(sid * _NPT, _NPT)])

        @pl.when(sid == _NS - 1)
        def _():
            pltpu.sync_copy(z_hbm.at[pl.ds(0, N - (_NS - 1) * _NPT)],
                            acc.at[pl.ds((_NS - 1) * _NPT,
                                         N - (_NS - 1) * _NPT)])

        plsc.subcore_barrier()

        def body(g, _):
            u0 = g * _GRPS
            pltpu.sync_copy(vals_hbm.at[pl.ds((ru + u0) * 128, _GRPS * 128)],
                            vbuf)
            for j in range(_GRPS):
                pltpu.sync_copy(vbuf.at[pl.ds(j * 128, 128)],
                                acc.at[idxbuf.at[u0 + j]], add=True)
            return 0

        lax.fori_loop(0, nu // _GRPS, body, 0)
        plsc.subcore_barrier()

        @pl.when(sid < _NS - 1)
        def _():
            pltpu.sync_copy(acc.at[pl.ds(sid * _NPT, _NPT)],
                            out_hbm.at[pl.ds(cid * N + sid * _NPT, _NPT)])

        @pl.when(sid == _NS - 1)
        def _():
            pltpu.sync_copy(
                acc.at[pl.ds((_NS - 1) * _NPT, N - (_NS - 1) * _NPT)],
                out_hbm.at[pl.ds(cid * N + (_NS - 1) * _NPT,
                                 N - (_NS - 1) * _NPT)])

    return k(vals, idx2d, zrows).reshape(2, N, P)


def _act(a, act):
    if act is None:
        return a
    if act == "relu":
        return jnp.maximum(a, 0.0)
    if act == "softmax":
        m = jnp.max(a, axis=-1, keepdims=True)
        e = jnp.exp(a - m)
        return e / jnp.sum(e, axis=-1, keepdims=True)
    if act == "sigmoid":
        return 1.0 / (1.0 + jnp.exp(-a))
    raise ValueError(act)


def _tc_linear(x, W, b, act=None, bm=_BM):
    M, K = x.shape
    P = W.shape[1]

    def kern(x_ref, w_ref, b_ref, o_ref):
        a = jnp.dot(x_ref[...], w_ref[...], preferred_element_type=jnp.float32)
        o_ref[...] = _act(a + b_ref[...], act)

    return pl.pallas_call(
        kern,
        grid=(pl.cdiv(M, bm),),
        in_specs=[
            pl.BlockSpec((bm, K), lambda i: (i, 0)),
            pl.BlockSpec((K, P), lambda i: (0, 0)),
            pl.BlockSpec((1, P), lambda i: (0, 0)),
        ],
        out_specs=pl.BlockSpec((bm, P), lambda i: (i, 0)),
        out_shape=jax.ShapeDtypeStruct((M, P), jnp.float32),
    )(x, W, b.reshape(1, P))


def _alpha_call(xls, xrd, ea, We, att):
    bm = _BM
    attf = att.reshape(1, D)

    def kern(xls_ref, xrd_ref, ea_ref, we_ref, att_ref, a_ref, g_ref):
        i = pl.program_id(0)
        m = xls_ref[...] + xrd_ref[...] + jnp.dot(
            ea_ref[...], we_ref[...], preferred_element_type=jnp.float32)
        m = jnp.where(m > 0, m, 0.2 * m) * att_ref[...]
        colh = lax.broadcasted_iota(jnp.int32, (D, H), 0) // C
        hh = lax.broadcasted_iota(jnp.int32, (D, H), 1)
        S = (colh == hh).astype(jnp.float32)
        a = jnp.dot(m, S, preferred_element_type=jnp.float32)
        a_ref[...] = a
        gm = jnp.max(a, axis=0)
        g2 = jnp.concatenate([gm, gm]).reshape(1, 2 * H)

        @pl.when(i == 0)
        def _():
            g_ref[...] = g2

        g_ref[...] = jnp.maximum(g_ref[...], g2)

    return pl.pallas_call(
        kern,
        grid=(pl.cdiv(E, bm),),
        in_specs=[
            pl.BlockSpec((bm, D), lambda i: (i, 0)),
            pl.BlockSpec((bm, D), lambda i: (i, 0)),
            pl.BlockSpec((bm, C), lambda i: (i, 0)),
            pl.BlockSpec((C, D), lambda i: (0, 0)),
            pl.BlockSpec((1, D), lambda i: (0, 0)),
        ],
        out_specs=[
            pl.BlockSpec((bm, H), lambda i: (i, 0)),
            pl.BlockSpec((1, 2 * H), lambda i: (0, 0)),
        ],
        out_shape=[
            jax.ShapeDtypeStruct((E, H), jnp.float32),
            jax.ShapeDtypeStruct((1, 2 * H), jnp.float32),
        ],
    )(xls, xrd, ea, We, attf)


def _msgw_call(xls, alpha, gmax):
    bm = _BM

    def kern(xls_ref, a_ref, g_ref, o_ref):
        w = jnp.exp(a_ref[...] - g_ref[...][:, :H])
        colh = lax.broadcasted_iota(jnp.int32, (H, D), 1) // C
        hh = lax.broadcasted_iota(jnp.int32, (H, D), 0)
        ST = (colh == hh).astype(jnp.float32)
        wexp = jnp.dot(w, ST, preferred_element_type=jnp.float32)
        o_ref[...] = xls_ref[...] * wexp

    return pl.pallas_call(
        kern,
        grid=(pl.cdiv(E, bm),),
        in_specs=[
            pl.BlockSpec((bm, D), lambda i: (i, 0)),
            pl.BlockSpec((bm, H), lambda i: (i, 0)),
            pl.BlockSpec((1, 2 * H), lambda i: (0, 0)),
        ],
        out_specs=pl.BlockSpec((bm, D), lambda i: (i, 0)),
        out_shape=jax.ShapeDtypeStruct((E, D), jnp.float32),
    )(xls, alpha, gmax)


_AROW = 640


def _sc_scatter_heads(alpha_flat, g16, idx2d, z128, rowiota):
    mesh = plsc.VectorSubcoreMesh(core_axis_name="c", subcore_axis_name="s")

    @functools.partial(
        pl.kernel,
        out_type=jax.ShapeDtypeStruct((_NC, N * H // 128, 128), jnp.float32),
        mesh=mesh,
        compiler_params=pltpu.CompilerParams(needs_layout_passes=False),
        scratch_types=[
            pltpu.VMEM((_MAXU, 128), jnp.int32),
            pltpu.VMEM((_GRP * 128 * H,), jnp.float32),
            pltpu.VMEM((16,), jnp.float32),
            pltpu.VMEM((_AROW, 128), jnp.float32),
            pltpu.VMEM((_AROW // 128, 128), jnp.int32),
            pltpu.VMEM_SHARED((_AROW, 128), jnp.float32),
            pltpu.SemaphoreType.DMA,
        ],
    )
    def k(a_hbm, g_hbm, idx_hbm, z_hbm, ri_hbm, out_hbm, idxbuf, abuf, gbuf,
          acc, riota, shacc, sem):
        cid = lax.axis_index("c")
        sid = lax.axis_index("s")
        w = sid * _NC + cid
        nu, ru = _worker_span(w)
        _load_idx(idx_hbm, idxbuf, w, ru)
        pltpu.sync_copy(z_hbm, acc)
        nz = _AROW // _NS
        pltpu.sync_copy(z_hbm.at[pl.ds(0, nz)],
                        shacc.at[pl.ds(sid * nz, nz)])
        pltpu.sync_copy(ri_hbm, riota)
        pltpu.sync_copy(g_hbm, gbuf)
        io = lax.iota(jnp.int32, 16)
        mlo = io < 8
        mhi = jnp.logical_not(mlo)

        def gblk(g, _):
            u0 = g * _GRP
            pltpu.sync_copy(
                a_hbm.at[pl.ds((ru + u0) * 128 * H, _GRP * 128 * H)], abuf)

            def unit(uj, _1):

                def grp(j16, _2):
                    dvec = idxbuf[u0 + uj, pl.ds(j16 * 16, 16)]
                    for p in range(8):
                        ex = jnp.exp(
                            abuf[pl.ds((uj * 64 + j16 * 8 + p) * 16, 16)]
                            - gbuf[...])
                        d0 = dvec[2 * p]
                        d1 = dvec[2 * p + 1]
                        addr = jnp.where(mlo, d0 * H + io, d1 * H + (io - 8))
                        arow = lax.shift_right_logical(addr, 7)
                        acol = jnp.bitwise_and(addr, 127)
                        plsc.addupdate_scatter(acc, [arow, acol], ex,
                                               mask=mlo)
                        plsc.addupdate_scatter(acc, [arow, acol], ex,
                                               mask=mhi)
                    return _2

                lax.fori_loop(0, 8, grp, 0)
                return _1

            lax.fori_loop(0, _GRP, unit, 0)
            return _

        lax.fori_loop(0, nu // _GRP, gblk, 0)
        plsc.subcore_barrier()

        def red(kk, _):
            pltpu.sync_copy(acc.at[pl.ds(kk * 128, 128)],
                            shacc.at[riota.at[kk]], add=True)
            return 0

        lax.fori_loop(0, _AROW // 128, red, 0)
        plsc.subcore_barrier()

        @pl.when(sid < _NS - 1)
        def _():
            pltpu.sync_copy(shacc.at[pl.ds(sid * 40, 40)],
                            out_hbm.at[cid, pl.ds(sid * 40, 40)])

        @pl.when(sid == _NS - 1)
        def _():
            pltpu.sync_copy(shacc.at[pl.ds(600, 25)],
                            out_hbm.at[cid, pl.ds(600, 25)])

    out = k(alpha_flat, g16, idx2d, z128, rowiota)
    return out.reshape(_NC, N, H)


def _combine_ln_call(parts, dparts, bias, g, b, res=None):
    P = parts.shape[0]
    PD = dparts.shape[0]
    bn = 2048
    have_res = res is not None

    def kern(*refs):
        if have_res:
            p_ref, dp_ref, bias_ref, g_ref, b_ref, res_ref, o_ref = refs
        else:
            p_ref, dp_ref, bias_ref, g_ref, b_ref, o_ref = refs
        den = jnp.sum(dp_ref[...], axis=0)
        rec = 1.0 / (den + 1e-16)
        colh = lax.broadcasted_iota(jnp.int32, (H, D), 1) // C
        hh = lax.broadcasted_iota(jnp.int32, (H, D), 0)
        ST = (colh == hh).astype(jnp.float32)
        recx = jnp.dot(rec, ST, preferred_element_type=jnp.float32)
        hsum = jnp.sum(p_ref[...], axis=0) * recx + bias_ref[...]
        mu = jnp.mean(hsum, axis=-1, keepdims=True)
        var = jnp.mean((hsum - mu) ** 2, axis=-1, keepdims=True)
        hn = (hsum - mu) / jnp.sqrt(var + 1e-5) * g_ref[...] + b_ref[...]
        hn = jnp.maximum(hn, 0.0)
        if have_res:
            hn = hn + res_ref[...]
        o_ref[...] = hn

    in_specs = [
        pl.BlockSpec((P, bn, D), lambda i: (0, i, 0)),
        pl.BlockSpec((PD, bn, H), lambda i: (0, i, 0)),
        pl.BlockSpec((1, D), lambda i: (0, 0)),
        pl.BlockSpec((1, D), lambda i: (0, 0)),
        pl.BlockSpec((1, D), lambda i: (0, 0)),
    ]
    args = [parts, dparts, bias.reshape(1, D), g.reshape(1, D),
            b.reshape(1, D)]
    if have_res:
        in_specs.append(pl.BlockSpec((bn, D), lambda i: (i, 0)))
        args.append(res)
    return pl.pallas_call(
        kern,
        grid=(pl.cdiv(N, bn),),
        in_specs=in_specs,
        out_specs=pl.BlockSpec((bn, D), lambda i: (i, 0)),
        out_shape=jax.ShapeDtypeStruct((N, D), jnp.float32),
    )(*args)


def _edge_head_call(hs, hd, Weh, beh, Wm1, bm1, Wm2, bm2):
    bm = _BM
    Wm1a = Wm1[:D]
    Wm1b = Wm1[D:]

    def kern(hs_ref, hd_ref, weh_ref, beh_ref, w1a_ref, w1b_ref, b1_ref,
             w2_ref, b2_ref, et_ref, ep_ref):
        hs_ = hs_ref[...]
        hd_ = hd_ref[...]
        et = jnp.dot(hs_, weh_ref[...], preferred_element_type=jnp.float32)
        et_ref[...] = _act(et + beh_ref[...], "softmax")
        hid = jnp.dot(hs_, w1a_ref[...], preferred_element_type=jnp.float32)
        hid = hid + jnp.dot(hd_, w1b_ref[...], preferred_element_type=jnp.float32)
        hid = jnp.maximum(hid + b1_ref[...], 0.0)
        ep = jnp.dot(hid, w2_ref[...], preferred_element_type=jnp.float32)
        ep_ref[...] = _act(ep + b2_ref[...], "sigmoid")

    return pl.pallas_call(
        kern,
        grid=(pl.cdiv(E, bm),),
        in_specs=[
            pl.BlockSpec((bm, D), lambda i: (i, 0)),
            pl.BlockSpec((bm, D), lambda i: (i, 0)),
            pl.BlockSpec((D, 6), lambda i: (0, 0)),
            pl.BlockSpec((1, 6), lambda i: (0, 0)),
            pl.BlockSpec((D, D), lambda i: (0, 0)),
            pl.BlockSpec((D, D), lambda i: (0, 0)),
            pl.BlockSpec((1, D), lambda i: (0, 0)),
            pl.BlockSpec((D, 1), lambda i: (0, 0)),
            pl.BlockSpec((1, 1), lambda i: (0, 0)),
        ],
        out_specs=[
            pl.BlockSpec((bm, 6), lambda i: (i, 0)),
            pl.BlockSpec((bm, 1), lambda i: (i, 0)),
        ],
        out_shape=[
            jax.ShapeDtypeStruct((E, 6), jnp.float32),
            jax.ShapeDtypeStruct((E, 1), jnp.float32),
        ],
    )(hs, hd, Weh, beh.reshape(1, 6), Wm1a, Wm1b, bm1.reshape(1, D),
      Wm2, bm2.reshape(1, 1))


def _gat_layer(h, src2d, dst2d, ea, Wl, bl, Wr, br, We, att, bias, g, bln,
               res, rowiota, z128):
    xl = _tc_linear(h, Wl, bl)
    xr = _tc_linear(h, Wr, br)
    xls = _sc_gather(xl, src2d, D)
    xrd = _sc_gather(xr, dst2d, D)
    alpha, gmax = _alpha_call(xls, xrd, ea, We, att)
    denom_parts = _sc_scatter_heads(alpha.reshape(-1), gmax.reshape(-1),
                                    dst2d, z128, rowiota)
    msg = _msgw_call(xls, alpha, gmax)
    out_parts = _sc_scatter(msg, dst2d, z128, D)
    return _combine_ln_call(out_parts, denom_parts, bias, g, bln, res)


def kernel(x, edge_features, edge_index, Wn, bn, Wet, bet, Wl1, bl1, Wr1, br1,
           We1, att1, bias1, g1, b1, Wl2, bl2, Wr2, br2, We2, att2, bias2, g2,
           b2, Wnh, bnh, Weh, beh, Wm1, bm1, Wm2, bm2):
    src2d = edge_index[0].reshape(_UNITS, 128)
    dst2d = edge_index[1].reshape(_UNITS, 128)
    rowiota = jnp.arange(_AROW, dtype=jnp.int32).reshape(_AROW // 128, 128)
    z128 = jnp.zeros((_NPT, D), jnp.float32)
    ea = _tc_linear(edge_features, Wet, bet)
    h0 = _tc_linear(x, Wn, bn)
    h1 = _gat_layer(h0, src2d, dst2d, ea, Wl1, bl1, Wr1, br1, We1, att1,
                    bias1, g1, b1, None, rowiota, z128)
    h = _gat_layer(h1, src2d, dst2d, ea, Wl2, bl2, Wr2, br2, We2, att2,
                   bias2, g2, b2, h0, rowiota, z128)
    node_type_preds = _tc_linear(h, Wnh, bnh, act="softmax")
    hs = _sc_gather(h, src2d, D)
    hd = _sc_gather(h, dst2d, D)
    edge_type_preds, edge_existence_preds = _edge_head_call(
        hs, hd, Weh, beh, Wm1, bm1, Wm2, bm2)
    return node_type_preds, edge_type_preds, edge_existence_preds


# speedup vs baseline: 22.4776x; 1.1768x over previous
import functools

import jax
import jax.numpy as jnp
from jax import lax
from jax.experimental import pallas as pl
from jax.experimental.pallas import tpu as pltpu
from jax.experimental.pallas import tpu_sc as plsc

N = 10000
E = 320000
H = 8
C = 16
D = 128

_BM = 512

_NC = 2
_NS = 16
_NW = _NC * _NS
_UNITS = E // 128
_WHI = 24
_U_HI = 80
_U_LO = 72
_TAIL = _UNITS - (_WHI * _U_HI + (_NW - _WHI) * _U_LO)
_MAXU = _U_HI
_GRP = 4
_GRPS = 2
_NPT = 640


def _worker_span(w):
    nu = jnp.where(w < _WHI, _U_HI, _U_LO) + jnp.where(w == _NW - 1, _TAIL, 0)
    ru = jnp.where(w < _WHI, _U_HI * w, _WHI * _U_HI + _U_LO * (w - _WHI))
    return nu, ru


def _load_idx(idx_hbm, idxbuf, w, ru):
    pltpu.sync_copy(idx_hbm.at[pl.ds(ru, _U_LO)], idxbuf.at[pl.ds(0, _U_LO)])

    @pl.when(w < _WHI)
    def _():
        pltpu.sync_copy(idx_hbm.at[pl.ds(ru + _U_LO, _U_HI - _U_LO)],
                        idxbuf.at[pl.ds(_U_LO, _U_HI - _U_LO)])

    @pl.when(w == _NW - 1)
    def _():
        pltpu.sync_copy(idx_hbm.at[pl.ds(_UNITS - _TAIL, _TAIL)],
                        idxbuf.at[pl.ds(_U_LO, _TAIL)])


def _sc_gather(table, idx2d, P):
    mesh = plsc.VectorSubcoreMesh(core_axis_name="c", subcore_axis_name="s")

    @functools.partial(
        pl.kernel,
        out_type=jax.ShapeDtypeStruct((E, P), jnp.float32),
        mesh=mesh,
        scratch_types=[
            pltpu.VMEM((_MAXU, 128), jnp.int32),
            pltpu.VMEM((_GRP * 128, P), jnp.float32),
            pltpu.SemaphoreType.DMA,
        ],
    )
    def k(table_hbm, idx_hbm, out_hbm, idxbuf, rows, sem):
        w = lax.axis_index("s") * _NC + lax.axis_index("c")
        nu, ru = _worker_span(w)
        _load_idx(idx_hbm, idxbuf, w, ru)

        def body(g, _):
            u0 = g * _GRP
            cps = [
                pltpu.async_copy(table_hbm.at[idxbuf.at[u0 + j]],
                                 rows.at[pl.ds(j * 128, 128)], sem)
                for j in range(_GRP)
            ]
            for cp in cps:
                cp.wait()
            pltpu.sync_copy(rows,
                            out_hbm.at[pl.ds((ru + u0) * 128, _GRP * 128)])
            return 0

        lax.fori_loop(0, nu // _GRP, body, 0)

    return k(table, idx2d)


def _sc_scatter(vals, idx2d, zrows, P):
    mesh = plsc.VectorSubcoreMesh(core_axis_name="c", subcore_axis_name="s")

    @functools.partial(
        pl.kernel,
        out_type=jax.ShapeDtypeStruct((2 * N, P), jnp.float32),
        mesh=mesh,
        scratch_types=[
            pltpu.VMEM((_MAXU, 128), jnp.int32),
            pltpu.VMEM((_GRPS * 128, P), jnp.float32),
            pltpu.VMEM_SHARED((N, P), jnp.float32),
            pltpu.SemaphoreType.DMA,
        ],
    )
    def k(vals_hbm, idx_hbm, z_hbm, out_hbm, idxbuf, vbuf, acc, sem):
        cid = lax.axis_index("c")
        sid = lax.axis_index("s")
        w = sid * _NC + cid
        nu, ru = _worker_span(w)
        _load_idx(idx_hbm, idxbuf, w, ru)

        @pl.when(sid < _NS - 1)
        def _():
            pltpu.sync_copy(z_hbm, acc.at[pl.ds(sid * _NPT, _NPT)])

        @pl.when(sid == _NS - 1)
        def _():
            pltpu.sync_copy(z_hbm.at[pl.ds(0, N - (_NS - 1) * _NPT)],
                            acc.at[pl.ds((_NS - 1) * _NPT,
                                         N - (_NS - 1) * _NPT)])

        plsc.subcore_barrier()

        def body(g, _):
            u0 = g * _GRPS
            pltpu.sync_copy(vals_hbm.at[pl.ds((ru + u0) * 128, _GRPS * 128)],
                            vbuf)
            for j in range(_GRPS):
                pltpu.sync_copy(vbuf.at[pl.ds(j * 128, 128)],
                                acc.at[idxbuf.at[u0 + j]], add=True)
            return 0

        lax.fori_loop(0, nu // _GRPS, body, 0)
        plsc.subcore_barrier()

        @pl.when(sid < _NS - 1)
        def _():
            pltpu.sync_copy(acc.at[pl.ds(sid * _NPT, _NPT)],
                            out_hbm.at[pl.ds(cid * N + sid * _NPT, _NPT)])

        @pl.when(sid == _NS - 1)
        def _():
            pltpu.sync_copy(
                acc.at[pl.ds((_NS - 1) * _NPT, N - (_NS - 1) * _NPT)],
                out_hbm.at[pl.ds(cid * N + (_NS - 1) * _NPT,
                                 N - (_NS - 1) * _NPT)])

    return k(vals, idx2d, zrows).reshape(2, N, P)


def _act(a, act):
    if act is None:
        return a
    if act == "relu":
        return jnp.maximum(a, 0.0)
    if act == "softmax":
        m = jnp.max(a, axis=-1, keepdims=True)
        e = jnp.exp(a - m)
        return e / jnp.sum(e, axis=-1, keepdims=True)
    if act == "sigmoid":
        return 1.0 / (1.0 + jnp.exp(-a))
    raise ValueError(act)


def _tc_linear(x, W, b, act=None, bm=_BM):
    M, K = x.shape
    P = W.shape[1]

    def kern(x_ref, w_ref, b_ref, o_ref):
        a = jnp.dot(x_ref[...], w_ref[...], preferred_element_type=jnp.float32)
        o_ref[...] = _act(a + b_ref[...], act)

    return pl.pallas_call(
        kern,
        grid=(pl.cdiv(M, bm),),
        in_specs=[
            pl.BlockSpec((bm, K), lambda i: (i, 0)),
            pl.BlockSpec((K, P), lambda i: (0, 0)),
            pl.BlockSpec((1, P), lambda i: (0, 0)),
        ],
        out_specs=pl.BlockSpec((bm, P), lambda i: (i, 0)),
        out_shape=jax.ShapeDtypeStruct((M, P), jnp.float32),
    )(x, W, b.reshape(1, P))


def _alpha_call(xls, xrd, ea, We, att):
    bm = _BM

    def kern(xls_ref, xrd_ref, ea_ref, we_ref, att_ref, ex_ref, msg_ref):
        xls_ = xls_ref[...]
        m = xls_ + xrd_ref[...] + jnp.dot(
            ea_ref[...], we_ref[...], preferred_element_type=jnp.float32)
        m = jnp.where(m > 0, m, 0.2 * m) * att_ref[...]
        colh = lax.broadcasted_iota(jnp.int32, (D, H), 0) // C
        hh = lax.broadcasted_iota(jnp.int32, (D, H), 1)
        S = (colh == hh).astype(jnp.float32)
        ex = jnp.exp(jnp.dot(m, S, preferred_element_type=jnp.float32))
        ex_ref[...] = ex
        exx = jnp.dot(ex, S.T, preferred_element_type=jnp.float32)
        msg_ref[...] = xls_ * exx

    return pl.pallas_call(
        kern,
        grid=(pl.cdiv(E, bm),),
        in_specs=[
            pl.BlockSpec((bm, D), lambda i: (i, 0)),
            pl.BlockSpec((bm, D), lambda i: (i, 0)),
            pl.BlockSpec((bm, C), lambda i: (i, 0)),
            pl.BlockSpec((C, D), lambda i: (0, 0)),
            pl.BlockSpec((1, D), lambda i: (0, 0)),
        ],
        out_specs=[
            pl.BlockSpec((bm, H), lambda i: (i, 0)),
            pl.BlockSpec((bm, D), lambda i: (i, 0)),
        ],
        out_shape=[
            jax.ShapeDtypeStruct((E, H), jnp.float32),
            jax.ShapeDtypeStruct((E, D), jnp.float32),
        ],
    )(xls, xrd, ea, We, att.reshape(1, D))


_AROW = 640


def _sc_scatter_heads(ex_flat, idx2d, z128):
    mesh = plsc.VectorSubcoreMesh(core_axis_name="c", subcore_axis_name="s")

    @functools.partial(
        pl.kernel,
        out_type=jax.ShapeDtypeStruct((_NW, N * H // 128, 128), jnp.float32),
        mesh=mesh,
        compiler_params=pltpu.CompilerParams(needs_layout_passes=False),
        scratch_types=[
            pltpu.VMEM((_MAXU, 128), jnp.int32),
            pltpu.VMEM((_GRP * 128 * H,), jnp.float32),
            pltpu.VMEM((_AROW, 128), jnp.float32),
            pltpu.SemaphoreType.DMA,
        ],
    )
    def k(a_hbm, idx_hbm, z_hbm, out_hbm, idxbuf, abuf, acc, sem):
        w = lax.axis_index("s") * _NC + lax.axis_index("c")
        nu, ru = _worker_span(w)
        _load_idx(idx_hbm, idxbuf, w, ru)
        pltpu.sync_copy(z_hbm, acc)
        io = lax.iota(jnp.int32, 16)
        mlo = io < 8
        mhi = jnp.logical_not(mlo)

        def gblk(g, _):
            u0 = g * _GRP
            pltpu.sync_copy(
                a_hbm.at[pl.ds((ru + u0) * 128 * H, _GRP * 128 * H)], abuf)

            def unit(uj, _1):

                def grp(j16, _2):
                    dvec = idxbuf[u0 + uj, pl.ds(j16 * 16, 16)]
                    for p in range(8):
                        ex = abuf[pl.ds((uj * 64 + j16 * 8 + p) * 16, 16)]
                        d0 = dvec[2 * p]
                        d1 = dvec[2 * p + 1]
                        addr = jnp.where(mlo, d0 * H + io, d1 * H + (io - 8))
                        arow = lax.shift_right_logical(addr, 7)
                        acol = jnp.bitwise_and(addr, 127)
                        plsc.addupdate_scatter(acc, [arow, acol], ex,
                                               mask=mlo)
                        plsc.addupdate_scatter(acc, [arow, acol], ex,
                                               mask=mhi)
                    return _2

                lax.fori_loop(0, 8, grp, 0)
                return _1

            lax.fori_loop(0, _GRP, unit, 0)
            return _

        lax.fori_loop(0, nu // _GRP, gblk, 0)
        pltpu.sync_copy(acc.at[pl.ds(0, N * H // 128)], out_hbm.at[w])

    return k(ex_flat, idx2d, z128)


def _recpack_call(dparts):
    R = N * H // 128
    bn = 128

    def kern(dp_ref, o_ref):
        o_ref[...] = 1.0 / (jnp.sum(dp_ref[...], axis=0) + 1e-16)

    return pl.pallas_call(
        kern,
        grid=(pl.cdiv(R, bn),),
        in_specs=[pl.BlockSpec((_NW, bn, 128), lambda i: (0, i, 0))],
        out_specs=pl.BlockSpec((bn, 128), lambda i: (i, 0)),
        out_shape=jax.ShapeDtypeStruct((R, 128), jnp.float32),
    )(dparts)


def _combine_ln_call(parts, rec, bias, g, b, res=None):
    P = parts.shape[0]
    bn = 2048
    have_res = res is not None

    def kern(*refs):
        if have_res:
            p_ref, rec_ref, bias_ref, g_ref, b_ref, res_ref, o_ref = refs
        else:
            p_ref, rec_ref, bias_ref, g_ref, b_ref, o_ref = refs
        colh = lax.broadcasted_iota(jnp.int32, (H, D), 1) // C
        hh = lax.broadcasted_iota(jnp.int32, (H, D), 0)
        ST = (colh == hh).astype(jnp.float32)
        recx = jnp.dot(rec_ref[...], ST, preferred_element_type=jnp.float32)
        hsum = jnp.sum(p_ref[...], axis=0) * recx + bias_ref[...]
        mu = jnp.mean(hsum, axis=-1, keepdims=True)
        var = jnp.mean((hsum - mu) ** 2, axis=-1, keepdims=True)
        hn = (hsum - mu) / jnp.sqrt(var + 1e-5) * g_ref[...] + b_ref[...]
        hn = jnp.maximum(hn, 0.0)
        if have_res:
            hn = hn + res_ref[...]
        o_ref[...] = hn

    in_specs = [
        pl.BlockSpec((P, bn, D), lambda i: (0, i, 0)),
        pl.BlockSpec((bn, H), lambda i: (i, 0)),
        pl.BlockSpec((1, D), lambda i: (0, 0)),
        pl.BlockSpec((1, D), lambda i: (0, 0)),
        pl.BlockSpec((1, D), lambda i: (0, 0)),
    ]
    args = [parts, rec, bias.reshape(1, D), g.reshape(1, D),
            b.reshape(1, D)]
    if have_res:
        in_specs.append(pl.BlockSpec((bn, D), lambda i: (i, 0)))
        args.append(res)
    return pl.pallas_call(
        kern,
        grid=(pl.cdiv(N, bn),),
        in_specs=in_specs,
        out_specs=pl.BlockSpec((bn, D), lambda i: (i, 0)),
        out_shape=jax.ShapeDtypeStruct((N, D), jnp.float32),
    )(*args)


def _edge_head_call(hs, hd, Weh, beh, Wm1, bm1, Wm2, bm2):
    bm = _BM
    Wm1a = Wm1[:D]
    Wm1b = Wm1[D:]

    def kern(hs_ref, hd_ref, weh_ref, beh_ref, w1a_ref, w1b_ref, b1_ref,
             w2_ref, b2_ref, et_ref, ep_ref):
        hs_ = hs_ref[...]
        hd_ = hd_ref[...]
        et = jnp.dot(hs_, weh_ref[...], preferred_element_type=jnp.float32)
        et_ref[...] = _act(et + beh_ref[...], "softmax")
        hid = jnp.dot(hs_, w1a_ref[...], preferred_element_type=jnp.float32)
        hid = hid + jnp.dot(hd_, w1b_ref[...], preferred_element_type=jnp.float32)
        hid = jnp.maximum(hid + b1_ref[...], 0.0)
        ep = jnp.dot(hid, w2_ref[...], preferred_element_type=jnp.float32)
        ep_ref[...] = _act(ep + b2_ref[...], "sigmoid")

    return pl.pallas_call(
        kern,
        grid=(pl.cdiv(E, bm),),
        in_specs=[
            pl.BlockSpec((bm, D), lambda i: (i, 0)),
            pl.BlockSpec((bm, D), lambda i: (i, 0)),
            pl.BlockSpec((D, 6), lambda i: (0, 0)),
            pl.BlockSpec((1, 6), lambda i: (0, 0)),
            pl.BlockSpec((D, D), lambda i: (0, 0)),
            pl.BlockSpec((D, D), lambda i: (0, 0)),
            pl.BlockSpec((1, D), lambda i: (0, 0)),
            pl.BlockSpec((D, 1), lambda i: (0, 0)),
            pl.BlockSpec((1, 1), lambda i: (0, 0)),
        ],
        out_specs=[
            pl.BlockSpec((bm, 6), lambda i: (i, 0)),
            pl.BlockSpec((bm, 1), lambda i: (i, 0)),
        ],
        out_shape=[
            jax.ShapeDtypeStruct((E, 6), jnp.float32),
            jax.ShapeDtypeStruct((E, 1), jnp.float32),
        ],
    )(hs, hd, Weh, beh.reshape(1, 6), Wm1a, Wm1b, bm1.reshape(1, D),
      Wm2, bm2.reshape(1, 1))


def _gat_layer(h, src2d, dst2d, ea, Wl, bl, Wr, br, We, att, bias, g, bln,
               res, z128):
    xl = _tc_linear(h, Wl, bl)
    xr = _tc_linear(h, Wr, br)
    xls = _sc_gather(xl, src2d, D)
    xrd = _sc_gather(xr, dst2d, D)
    ex, msg = _alpha_call(xls, xrd, ea, We, att)
    denom_parts = _sc_scatter_heads(ex.reshape(-1), dst2d, z128)
    rec = _recpack_call(denom_parts).reshape(N, H)
    out_parts = _sc_scatter(msg, dst2d, z128, D)
    return _combine_ln_call(out_parts, rec, bias, g, bln, res)


def kernel(x, edge_features, edge_index, Wn, bn, Wet, bet, Wl1, bl1, Wr1, br1,
           We1, att1, bias1, g1, b1, Wl2, bl2, Wr2, br2, We2, att2, bias2, g2,
           b2, Wnh, bnh, Weh, beh, Wm1, bm1, Wm2, bm2):
    src2d = edge_index[0].reshape(_UNITS, 128)
    dst2d = edge_index[1].reshape(_UNITS, 128)
    z128 = jnp.zeros((_NPT, D), jnp.float32)
    ea = _tc_linear(edge_features, Wet, bet)
    h0 = _tc_linear(x, Wn, bn)
    h1 = _gat_layer(h0, src2d, dst2d, ea, Wl1, bl1, Wr1, br1, We1, att1,
                    bias1, g1, b1, None, z128)
    h = _gat_layer(h1, src2d, dst2d, ea, Wl2, bl2, Wr2, br2, We2, att2,
                   bias2, g2, b2, h0, z128)
    node_type_preds = _tc_linear(h, Wnh, bnh, act="softmax")
    hs = _sc_gather(h, src2d, D)
    hd = _sc_gather(h, dst2d, D)
    edge_type_preds, edge_existence_preds = _edge_head_call(
        hs, hd, Weh, beh, Wm1, bm1, Wm2, bm2)
    return node_type_preds, edge_type_preds, edge_existence_preds


# speedup vs baseline: 22.6171x; 1.0062x over previous
import functools

import jax
import jax.numpy as jnp
from jax import lax
from jax.experimental import pallas as pl
from jax.experimental.pallas import tpu as pltpu
from jax.experimental.pallas import tpu_sc as plsc

N = 10000
E = 320000
H = 8
C = 16
D = 128

_BM = 512

_NC = 2
_NS = 16
_NW = _NC * _NS
_UNITS = E // 128
_WHI = 24
_U_HI = 80
_U_LO = 72
_TAIL = _UNITS - (_WHI * _U_HI + (_NW - _WHI) * _U_LO)
_MAXU = _U_HI
_GRP = 4
_GRPS = 2
_NPT = 640


def _worker_span(w):
    nu = jnp.where(w < _WHI, _U_HI, _U_LO) + jnp.where(w == _NW - 1, _TAIL, 0)
    ru = jnp.where(w < _WHI, _U_HI * w, _WHI * _U_HI + _U_LO * (w - _WHI))
    return nu, ru


def _load_idx(idx_hbm, idxbuf, w, ru):
    pltpu.sync_copy(idx_hbm.at[pl.ds(ru, _U_LO)], idxbuf.at[pl.ds(0, _U_LO)])

    @pl.when(w < _WHI)
    def _():
        pltpu.sync_copy(idx_hbm.at[pl.ds(ru + _U_LO, _U_HI - _U_LO)],
                        idxbuf.at[pl.ds(_U_LO, _U_HI - _U_LO)])

    @pl.when(w == _NW - 1)
    def _():
        pltpu.sync_copy(idx_hbm.at[pl.ds(_UNITS - _TAIL, _TAIL)],
                        idxbuf.at[pl.ds(_U_LO, _TAIL)])


def _sc_gather2(t1, idx1, t2, idx2, P):
    mesh = plsc.VectorSubcoreMesh(core_axis_name="c", subcore_axis_name="s")

    @functools.partial(
        pl.kernel,
        out_type=[
            jax.ShapeDtypeStruct((E, P), jnp.float32),
            jax.ShapeDtypeStruct((E, P), jnp.float32),
        ],
        mesh=mesh,
        scratch_types=[
            pltpu.VMEM((_MAXU, 128), jnp.int32),
            pltpu.VMEM((_MAXU, 128), jnp.int32),
            pltpu.VMEM((_GRP * 128, P), jnp.float32),
            pltpu.SemaphoreType.DMA,
        ],
    )
    def k(t1_hbm, i1_hbm, t2_hbm, i2_hbm, o1_hbm, o2_hbm, ib1, ib2, rows,
          sem):
        w = lax.axis_index("s") * _NC + lax.axis_index("c")
        nu, ru = _worker_span(w)
        _load_idx(i1_hbm, ib1, w, ru)
        _load_idx(i2_hbm, ib2, w, ru)

        def body(g, _):
            u0 = g * _GRP
            for tab, ib, out in ((t1_hbm, ib1, o1_hbm), (t2_hbm, ib2, o2_hbm)):
                cps = [
                    pltpu.async_copy(tab.at[ib.at[u0 + j]],
                                     rows.at[pl.ds(j * 128, 128)], sem)
                    for j in range(_GRP)
                ]
                for cp in cps:
                    cp.wait()
                pltpu.sync_copy(rows,
                                out.at[pl.ds((ru + u0) * 128, _GRP * 128)])
            return 0

        lax.fori_loop(0, nu // _GRP, body, 0)

    return k(t1, idx1, t2, idx2)


def _sc_scatter(vals, idx2d, zrows, P):
    mesh = plsc.VectorSubcoreMesh(core_axis_name="c", subcore_axis_name="s")

    @functools.partial(
        pl.kernel,
        out_type=jax.ShapeDtypeStruct((2 * N, P), jnp.float32),
        mesh=mesh,
        scratch_types=[
            pltpu.VMEM((_MAXU, 128), jnp.int32),
            pltpu.VMEM((_GRPS * 128, P), jnp.float32),
            pltpu.VMEM_SHARED((N, P), jnp.float32),
            pltpu.SemaphoreType.DMA,
        ],
    )
    def k(vals_hbm, idx_hbm, z_hbm, out_hbm, idxbuf, vbuf, acc, sem):
        cid = lax.axis_index("c")
        sid = lax.axis_index("s")
        w = sid * _NC + cid
        nu, ru = _worker_span(w)
        _load_idx(idx_hbm, idxbuf, w, ru)

        @pl.when(sid < _NS - 1)
        def _():
            pltpu.sync_copy(z_hbm, acc.at[pl.ds(sid * _NPT, _NPT)])

        @pl.when(sid == _NS - 1)
        def _():
            pltpu.sync_copy(z_hbm.at[pl.ds(0, N - (_NS - 1) * _NPT)],
                            acc.at[pl.ds((_NS - 1) * _NPT,
                                         N - (_NS - 1) * _NPT)])

        plsc.subcore_barrier()

        def body(g, _):
            u0 = g * _GRPS
            pltpu.sync_copy(vals_hbm.at[pl.ds((ru + u0) * 128, _GRPS * 128)],
                            vbuf)
            for j in range(_GRPS):
                pltpu.sync_copy(vbuf.at[pl.ds(j * 128, 128)],
                                acc.at[idxbuf.at[u0 + j]], add=True)
            return 0

        lax.fori_loop(0, nu // _GRPS, body, 0)
        plsc.subcore_barrier()

        @pl.when(sid < _NS - 1)
        def _():
            pltpu.sync_copy(acc.at[pl.ds(sid * _NPT, _NPT)],
                            out_hbm.at[pl.ds(cid * N + sid * _NPT, _NPT)])

        @pl.when(sid == _NS - 1)
        def _():
            pltpu.sync_copy(
                acc.at[pl.ds((_NS - 1) * _NPT, N - (_NS - 1) * _NPT)],
                out_hbm.at[pl.ds(cid * N + (_NS - 1) * _NPT,
                                 N - (_NS - 1) * _NPT)])

    return k(vals, idx2d, zrows).reshape(2, N, P)


def _act(a, act):
    if act is None:
        return a
    if act == "relu":
        return jnp.maximum(a, 0.0)
    if act == "softmax":
        m = jnp.max(a, axis=-1, keepdims=True)
        e = jnp.exp(a - m)
        return e / jnp.sum(e, axis=-1, keepdims=True)
    if act == "sigmoid":
        return 1.0 / (1.0 + jnp.exp(-a))
    raise ValueError(act)


def _tc_linear(x, W, b, act=None, bm=_BM):
    M, K = x.shape
    P = W.shape[1]

    def kern(x_ref, w_ref, b_ref, o_ref):
        a = jnp.dot(x_ref[...], w_ref[...], preferred_element_type=jnp.float32)
        o_ref[...] = _act(a + b_ref[...], act)

    return pl.pallas_call(
        kern,
        grid=(pl.cdiv(M, bm),),
        in_specs=[
            pl.BlockSpec((bm, K), lambda i: (i, 0)),
            pl.BlockSpec((K, P), lambda i: (0, 0)),
            pl.BlockSpec((1, P), lambda i: (0, 0)),
        ],
        out_specs=pl.BlockSpec((bm, P), lambda i: (i, 0)),
        out_shape=jax.ShapeDtypeStruct((M, P), jnp.float32),
    )(x, W, b.reshape(1, P))


def _tc_linear2(x, W1, b1, W2, b2, bm=_BM):
    M, K = x.shape
    P = W1.shape[1]

    def kern(x_ref, w1_ref, b1_ref, w2_ref, b2_ref, o1_ref, o2_ref):
        x_ = x_ref[...]
        o1_ref[...] = jnp.dot(
            x_, w1_ref[...], preferred_element_type=jnp.float32) + b1_ref[...]
        o2_ref[...] = jnp.dot(
            x_, w2_ref[...], preferred_element_type=jnp.float32) + b2_ref[...]

    return pl.pallas_call(
        kern,
        grid=(pl.cdiv(M, bm),),
        in_specs=[
            pl.BlockSpec((bm, K), lambda i: (i, 0)),
            pl.BlockSpec((K, P), lambda i: (0, 0)),
            pl.BlockSpec((1, P), lambda i: (0, 0)),
            pl.BlockSpec((K, P), lambda i: (0, 0)),
            pl.BlockSpec((1, P), lambda i: (0, 0)),
        ],
        out_specs=[
            pl.BlockSpec((bm, P), lambda i: (i, 0)),
            pl.BlockSpec((bm, P), lambda i: (i, 0)),
        ],
        out_shape=[
            jax.ShapeDtypeStruct((M, P), jnp.float32),
            jax.ShapeDtypeStruct((M, P), jnp.float32),
        ],
    )(x, W1, b1.reshape(1, P), W2, b2.reshape(1, P))


def _alpha_call(xls, xrd, ea, We, att):
    bm = _BM

    def kern(xls_ref, xrd_ref, ea_ref, we_ref, att_ref, ex_ref, msg_ref):
        xls_ = xls_ref[...]
        m = xls_ + xrd_ref[...] + jnp.dot(
            ea_ref[...], we_ref[...], preferred_element_type=jnp.float32)
        m = jnp.where(m > 0, m, 0.2 * m) * att_ref[...]
        colh = lax.broadcasted_iota(jnp.int32, (D, H), 0) // C
        hh = lax.broadcasted_iota(jnp.int32, (D, H), 1)
        S = (colh == hh).astype(jnp.float32)
        ex = jnp.exp(jnp.dot(m, S, preferred_element_type=jnp.float32))
        ex_ref[...] = ex
        exx = jnp.dot(ex, S.T, preferred_element_type=jnp.float32)
        msg_ref[...] = xls_ * exx

    return pl.pallas_call(
        kern,
        grid=(pl.cdiv(E, bm),),
        in_specs=[
            pl.BlockSpec((bm, D), lambda i: (i, 0)),
            pl.BlockSpec((bm, D), lambda i: (i, 0)),
            pl.BlockSpec((bm, C), lambda i: (i, 0)),
            pl.BlockSpec((C, D), lambda i: (0, 0)),
            pl.BlockSpec((1, D), lambda i: (0, 0)),
        ],
        out_specs=[
            pl.BlockSpec((bm, H), lambda i: (i, 0)),
            pl.BlockSpec((bm, D), lambda i: (i, 0)),
        ],
        out_shape=[
            jax.ShapeDtypeStruct((E, H), jnp.float32),
            jax.ShapeDtypeStruct((E, D), jnp.float32),
        ],
    )(xls, xrd, ea, We, att.reshape(1, D))


_AROW = 640


def _sc_scatter_heads(ex_flat, idx2d, z128):
    mesh = plsc.VectorSubcoreMesh(core_axis_name="c", subcore_axis_name="s")

    @functools.partial(
        pl.kernel,
        out_type=jax.ShapeDtypeStruct((_NW, N * H // 128, 128), jnp.float32),
        mesh=mesh,
        compiler_params=pltpu.CompilerParams(needs_layout_passes=False),
        scratch_types=[
            pltpu.VMEM((_MAXU, 128), jnp.int32),
            pltpu.VMEM((_GRP * 128 * H,), jnp.float32),
            pltpu.VMEM((_AROW, 128), jnp.float32),
            pltpu.SemaphoreType.DMA,
        ],
    )
    def k(a_hbm, idx_hbm, z_hbm, out_hbm, idxbuf, abuf, acc, sem):
        w = lax.axis_index("s") * _NC + lax.axis_index("c")
        nu, ru = _worker_span(w)
        _load_idx(idx_hbm, idxbuf, w, ru)
        pltpu.sync_copy(z_hbm, acc)
        io = lax.iota(jnp.int32, 16)
        mlo = io < 8
        mhi = jnp.logical_not(mlo)

        def gblk(g, _):
            u0 = g * _GRP
            pltpu.sync_copy(
                a_hbm.at[pl.ds((ru + u0) * 128 * H, _GRP * 128 * H)], abuf)

            def unit(uj, _1):

                def grp(j16, _2):
                    dvec = idxbuf[u0 + uj, pl.ds(j16 * 16, 16)]
                    for p in range(8):
                        ex = abuf[pl.ds((uj * 64 + j16 * 8 + p) * 16, 16)]
                        d0 = dvec[2 * p]
                        d1 = dvec[2 * p + 1]
                        addr = jnp.where(mlo, d0 * H + io, d1 * H + (io - 8))
                        arow = lax.shift_right_logical(addr, 7)
                        acol = jnp.bitwise_and(addr, 127)
                        plsc.addupdate_scatter(acc, [arow, acol], ex,
                                               mask=mlo)
                        plsc.addupdate_scatter(acc, [arow, acol], ex,
                                               mask=mhi)
                    return _2

                lax.fori_loop(0, 8, grp, 0)
                return _1

            lax.fori_loop(0, _GRP, unit, 0)
            return _

        lax.fori_loop(0, nu // _GRP, gblk, 0)
        pltpu.sync_copy(acc.at[pl.ds(0, N * H // 128)], out_hbm.at[w])

    return k(ex_flat, idx2d, z128)


def _recpack_call(dparts):
    R = N * H // 128
    bn = 128

    def kern(dp_ref, o_ref):
        o_ref[...] = 1.0 / (jnp.sum(dp_ref[...], axis=0) + 1e-16)

    return pl.pallas_call(
        kern,
        grid=(pl.cdiv(R, bn),),
        in_specs=[pl.BlockSpec((_NW, bn, 128), lambda i: (0, i, 0))],
        out_specs=pl.BlockSpec((bn, 128), lambda i: (i, 0)),
        out_shape=jax.ShapeDtypeStruct((R, 128), jnp.float32),
    )(dparts)


def _combine_ln_call(parts, rec, bias, g, b, res=None, head=None):
    P = parts.shape[0]
    bn = 2048
    have_res = res is not None
    have_head = head is not None

    def kern(*refs):
        refs = list(refs)
        p_ref, rec_ref, bias_ref, g_ref, b_ref = refs[:5]
        refs = refs[5:]
        res_ref = refs.pop(0) if have_res else None
        if have_head:
            wh_ref, bh_ref = refs.pop(0), refs.pop(0)
        o_ref = refs.pop(0)
        colh = lax.broadcasted_iota(jnp.int32, (H, D), 1) // C
        hh = lax.broadcasted_iota(jnp.int32, (H, D), 0)
        ST = (colh == hh).astype(jnp.float32)
        recx = jnp.dot(rec_ref[...], ST, preferred_element_type=jnp.float32)
        hsum = jnp.sum(p_ref[...], axis=0) * recx + bias_ref[...]
        mu = jnp.mean(hsum, axis=-1, keepdims=True)
        var = jnp.mean((hsum - mu) ** 2, axis=-1, keepdims=True)
        hn = (hsum - mu) / jnp.sqrt(var + 1e-5) * g_ref[...] + b_ref[...]
        hn = jnp.maximum(hn, 0.0)
        if have_res:
            hn = hn + res_ref[...]
        o_ref[...] = hn
        if have_head:
            nt = jnp.dot(hn, wh_ref[...], preferred_element_type=jnp.float32)
            refs.pop(0)[...] = _act(nt + bh_ref[...], "softmax")

    in_specs = [
        pl.BlockSpec((P, bn, D), lambda i: (0, i, 0)),
        pl.BlockSpec((bn, H), lambda i: (i, 0)),
        pl.BlockSpec((1, D), lambda i: (0, 0)),
        pl.BlockSpec((1, D), lambda i: (0, 0)),
        pl.BlockSpec((1, D), lambda i: (0, 0)),
    ]
    args = [parts, rec, bias.reshape(1, D), g.reshape(1, D),
            b.reshape(1, D)]
    if have_res:
        in_specs.append(pl.BlockSpec((bn, D), lambda i: (i, 0)))
        args.append(res)
    out_specs = pl.BlockSpec((bn, D), lambda i: (i, 0))
    out_shape = jax.ShapeDtypeStruct((N, D), jnp.float32)
    if have_head:
        Wh, bh = head
        in_specs.append(pl.BlockSpec((D, H), lambda i: (0, 0)))
        in_specs.append(pl.BlockSpec((1, H), lambda i: (0, 0)))
        args.append(Wh)
        args.append(bh.reshape(1, H))
        out_specs = [out_specs, pl.BlockSpec((bn, H), lambda i: (i, 0))]
        out_shape = [out_shape, jax.ShapeDtypeStruct((N, H), jnp.float32)]
    return pl.pallas_call(
        kern,
        grid=(pl.cdiv(N, bn),),
        in_specs=in_specs,
        out_specs=out_specs,
        out_shape=out_shape,
    )(*args)


def _edge_head_call(hs, hd, Weh, beh, Wm1, bm1, Wm2, bm2):
    bm = _BM
    Wm1a = Wm1[:D]
    Wm1b = Wm1[D:]

    def kern(hs_ref, hd_ref, weh_ref, beh_ref, w1a_ref, w1b_ref, b1_ref,
             w2_ref, b2_ref, et_ref, ep_ref):
        hs_ = hs_ref[...]
        hd_ = hd_ref[...]
        et = jnp.dot(hs_, weh_ref[...], preferred_element_type=jnp.float32)
        et_ref[...] = _act(et + beh_ref[...], "softmax")
        hid = jnp.dot(hs_, w1a_ref[...], preferred_element_type=jnp.float32)
        hid = hid + jnp.dot(hd_, w1b_ref[...], preferred_element_type=jnp.float32)
        hid = jnp.maximum(hid + b1_ref[...], 0.0)
        ep = jnp.dot(hid, w2_ref[...], preferred_element_type=jnp.float32)
        ep_ref[...] = _act(ep + b2_ref[...], "sigmoid")

    return pl.pallas_call(
        kern,
        grid=(pl.cdiv(E, bm),),
        in_specs=[
            pl.BlockSpec((bm, D), lambda i: (i, 0)),
            pl.BlockSpec((bm, D), lambda i: (i, 0)),
            pl.BlockSpec((D, 6), lambda i: (0, 0)),
            pl.BlockSpec((1, 6), lambda i: (0, 0)),
            pl.BlockSpec((D, D), lambda i: (0, 0)),
            pl.BlockSpec((D, D), lambda i: (0, 0)),
            pl.BlockSpec((1, D), lambda i: (0, 0)),
            pl.BlockSpec((D, 1), lambda i: (0, 0)),
            pl.BlockSpec((1, 1), lambda i: (0, 0)),
        ],
        out_specs=[
            pl.BlockSpec((bm, 6), lambda i: (i, 0)),
            pl.BlockSpec((bm, 1), lambda i: (i, 0)),
        ],
        out_shape=[
            jax.ShapeDtypeStruct((E, 6), jnp.float32),
            jax.ShapeDtypeStruct((E, 1), jnp.float32),
        ],
    )(hs, hd, Weh, beh.reshape(1, 6), Wm1a, Wm1b, bm1.reshape(1, D),
      Wm2, bm2.reshape(1, 1))


def _gat_layer(h, src2d, dst2d, ea, Wl, bl, Wr, br, We, att, bias, g, bln,
               res, z128, head=None):
    xl, xr = _tc_linear2(h, Wl, bl, Wr, br)
    xls, xrd = _sc_gather2(xl, src2d, xr, dst2d, D)
    ex, msg = _alpha_call(xls, xrd, ea, We, att)
    denom_parts = _sc_scatter_heads(ex.reshape(-1), dst2d, z128)
    rec = _recpack_call(denom_parts).reshape(N, H)
    out_parts = _sc_scatter(msg, dst2d, z128, D)
    return _combine_ln_call(out_parts, rec, bias, g, bln, res, head)


def kernel(x, edge_features, edge_index, Wn, bn, Wet, bet, Wl1, bl1, Wr1, br1,
           We1, att1, bias1, g1, b1, Wl2, bl2, Wr2, br2, We2, att2, bias2, g2,
           b2, Wnh, bnh, Weh, beh, Wm1, bm1, Wm2, bm2):
    src2d = edge_index[0].reshape(_UNITS, 128)
    dst2d = edge_index[1].reshape(_UNITS, 128)
    z128 = jnp.zeros((_NPT, D), jnp.float32)
    ea = _tc_linear(edge_features, Wet, bet)
    h0 = _tc_linear(x, Wn, bn)
    h1 = _gat_layer(h0, src2d, dst2d, ea, Wl1, bl1, Wr1, br1, We1, att1,
                    bias1, g1, b1, None, z128)
    h, node_type_preds = _gat_layer(h1, src2d, dst2d, ea, Wl2, bl2, Wr2,
                                    br2, We2, att2, bias2, g2, b2, h0, z128,
                                    head=(Wnh, bnh))
    hs, hd = _sc_gather2(h, src2d, h, dst2d, D)
    edge_type_preds, edge_existence_preds = _edge_head_call(
        hs, hd, Weh, beh, Wm1, bm1, Wm2, bm2)
    return node_type_preds, edge_type_preds, edge_existence_preds
